# baseline (speedup 1.0000x reference)
import functools

import jax
import jax.numpy as jnp
from jax import lax
from jax.experimental import pallas as pl
from jax.experimental.pallas import tpu as pltpu
from jax.experimental.pallas import tpu_sc as plsc

B = 16384
D = 32
V = 1_000_000
NC = 2
NS = 16
NW = NC * NS
BPW = B // NW
NB = 8
GRPS = BPW // 16
TAIL = (V // 128) * 128
TAIL_W = V - TAIL


def _fetch(tbl, rows_ref, l, slab, sem):
    j128 = pl.multiple_of((l >> 7) << 7, 128)
    pltpu.async_copy(tbl.at[rows_ref, pl.ds(j128, 128)], slab, sem)


def _body(label_hbm, mu_hbm, lv_hbm, out_mu, out_var, idx_v, rows_ref,
          *scratch):
    mu_slabs = scratch[0:NB]
    lv_slabs = scratch[NB:2 * NB]
    om_slab, ov_slab = scratch[2 * NB:2 * NB + 2]
    mu_sems = scratch[2 * NB + 2:3 * NB + 2]
    lv_sems = scratch[3 * NB + 2:4 * NB + 2]

    wid = lax.axis_index("s") * NC + lax.axis_index("c")
    base = wid * BPW

    pltpu.sync_copy(label_hbm.at[pl.ds(wid * GRPS, GRPS), :], idx_v)

    iota = lax.iota(jnp.int32, 16)
    rows_ref[pl.ds(0, 16)] = iota
    rows_ref[pl.ds(16, 16)] = iota + 16

    first = idx_v[0, :]
    for b in range(NB):
        _fetch(mu_hbm, rows_ref, first[b], mu_slabs[b], mu_sems[b])
        _fetch(lv_hbm, rows_ref, first[b], lv_slabs[b], lv_sems[b])

    def group(g, _):
        cur = idx_v[g, :]
        nxt = idx_v[jnp.minimum(g + 1, GRPS - 1), :]
        for k in range(16):
            i = g * 16 + k
            b = k % NB
            pltpu.make_async_copy(mu_hbm.at[:, pl.ds(0, 128)],
                                  mu_slabs[b], mu_sems[b]).wait()
            pltpu.make_async_copy(lv_hbm.at[:, pl.ds(0, 128)],
                                  lv_slabs[b], lv_sems[b]).wait()

            l = cur[k]
            lane = jnp.where(l >= TAIL, l - TAIL, l & 127)
            lane_v = jnp.full((16,), lane, jnp.int32)
            col_v = jnp.full((16,), i & 127, jnp.int32)
            for h in range(2):
                rows = iota + 16 * h
                v = plsc.load_gather(mu_slabs[b], [rows, lane_v])
                plsc.store_scatter(om_slab, [rows, col_v], v)
                w = plsc.load_gather(lv_slabs[b], [rows, lane_v])
                plsc.store_scatter(ov_slab, [rows, col_v], jnp.exp(w))

            if k == 15:
                @pl.when((g & 7) == 7)
                def _():
                    dst = pl.multiple_of(base + i - 127, 128)
                    pltpu.sync_copy(om_slab, out_mu.at[:, pl.ds(dst, 128)])
                    pltpu.sync_copy(ov_slab, out_var.at[:, pl.ds(dst, 128)])

            nl = cur[k + NB] if k + NB < 16 else nxt[k + NB - 16]

            @pl.when(i + NB < BPW)
            def _():
                _fetch(mu_hbm, rows_ref, nl, mu_slabs[b], mu_sems[b])
                _fetch(lv_hbm, rows_ref, nl, lv_slabs[b], lv_sems[b])
        return 0

    lax.fori_loop(0, GRPS, group, 0)


@jax.jit
def kernel(label, emb_mu, emb_logvar):
    mesh = plsc.VectorSubcoreMesh(core_axis_name="c", subcore_axis_name="s")
    slab = pltpu.VMEM((D, 128), jnp.float32)
    sem = pltpu.SemaphoreType.DMA
    f = functools.partial(
        pl.kernel,
        mesh=mesh,
        compiler_params=pltpu.CompilerParams(needs_layout_passes=False),
        out_type=[
            jax.ShapeDtypeStruct((D, B), jnp.float32),
            jax.ShapeDtypeStruct((D, B), jnp.float32),
        ],
        scratch_types=(
            [pltpu.VMEM((GRPS, 16), jnp.int32), pltpu.VMEM((D,), jnp.int32)]
            + [slab] * (2 * NB) + [slab] * 2 + [sem] * (2 * NB)
        ),
    )(_body)
    mu_t, var_t = f(label.astype(jnp.int32).reshape(B // 16, 16),
                    emb_mu.T, emb_logvar.T)
    return (mu_t.T, var_t.T)
